# baseline (speedup 1.0000x reference)
import jax
import jax.numpy as jnp
import numpy as np
from jax.experimental import pallas as pl

N = 8400
E = 42000
HEADS = 2


def _mm_kernel(x_ref, w_ref, o_ref):
    o_ref[...] = jnp.dot(x_ref[...], w_ref[...],
                         preferred_element_type=jnp.float32)


def _mm(x, w, bm=840, bn=1024):
    m, k = x.shape
    _, n = w.shape
    bn = min(bn, n)
    grid = (m // bm, n // bn)
    return pl.pallas_call(
        _mm_kernel,
        grid=grid,
        in_specs=[pl.BlockSpec((bm, k), lambda i, j: (i, 0)),
                  pl.BlockSpec((k, bn), lambda i, j: (0, j))],
        out_specs=pl.BlockSpec((bm, bn), lambda i, j: (i, j)),
        out_shape=jax.ShapeDtypeStruct((m, n), jnp.float32),
    )(x, w)


def _edge_stage(q, k, v, s_out, e_tab, src, dst, eid, H, C, n):
    e = e_tab[eid]
    kj = k[src] + e
    alpha = (q[dst] * kj).sum(-1) / np.sqrt(C).astype(np.float32)
    amax = jax.ops.segment_max(alpha, dst, num_segments=n)
    amax = jnp.where(jnp.isfinite(amax), amax, 0.0)
    ex = jnp.exp(alpha - amax[dst])
    den = jax.ops.segment_sum(ex, dst, num_segments=n)
    alpha = ex / (den[dst] + 1e-16)
    msg = (v[src] + e) * alpha[..., None]
    out = jax.ops.segment_sum(msg, dst, num_segments=n).reshape(n, H * C)
    return out + s_out


def kernel(x, edge_index, edge_features,
           Wq1, bq1, Wk1, bk1, Wv1, bv1, We1, Ws1, bs1,
           Wq2, bq2, Wk2, bk2, Wv2, bv2, We2, Ws2, bs2):
    src = edge_index[0]
    dst = edge_index[1]
    eid = jnp.arange(E, dtype=jnp.int32) % edge_features.shape[0]

    W1 = jnp.concatenate([Wq1, Wk1, Wv1, Ws1], axis=1)
    b1 = jnp.concatenate([bq1, bk1, bv1, bs1])
    qkvs = _mm(x, W1) + b1
    HC1 = Wq1.shape[1]
    q1 = qkvs[:, :HC1].reshape(N, HEADS, HC1 // HEADS)
    k1 = qkvs[:, HC1:2 * HC1].reshape(N, HEADS, HC1 // HEADS)
    v1 = qkvs[:, 2 * HC1:3 * HC1].reshape(N, HEADS, HC1 // HEADS)
    s1 = qkvs[:, 3 * HC1:]
    e1 = (edge_features @ We1).reshape(-1, HEADS, HC1 // HEADS)
    h = jax.nn.relu(_edge_stage(q1, k1, v1, s1, e1, src, dst, eid,
                                HEADS, HC1 // HEADS, N))

    W2 = jnp.concatenate([Wq2, Wk2, Wv2, Ws2], axis=1)
    b2 = jnp.concatenate([bq2, bk2, bv2, bs2])
    qkvs2 = _mm(h, W2, bn=256) + b2
    C2 = Wq2.shape[1]
    q2 = qkvs2[:, :C2].reshape(N, 1, C2)
    k2 = qkvs2[:, C2:2 * C2].reshape(N, 1, C2)
    v2 = qkvs2[:, 2 * C2:3 * C2].reshape(N, 1, C2)
    s2 = qkvs2[:, 3 * C2:]
    e2 = (edge_features @ We2).reshape(-1, 1, C2)
    h2 = jax.nn.relu(_edge_stage(q2, k2, v2, s2, e2, src, dst, eid,
                                 1, C2, N))
    return h2.reshape(-1, 420 * 64)


# speedup vs baseline: 1.1014x; 1.1014x over previous
import jax
import jax.numpy as jnp
import numpy as np
from jax.experimental import pallas as pl

N = 8400
E = 42000
NE = 2100
NC = N + NE
NCP = 10752


def _mm_kernel(x_ref, w_ref, o_ref):
    o_ref[...] = jnp.dot(x_ref[...].astype(jnp.bfloat16),
                         w_ref[...].astype(jnp.bfloat16),
                         preferred_element_type=jnp.float32)


def _mm(x, w, bm=840, bn=2048):
    m, k = x.shape
    _, n = w.shape
    bn = min(bn, n)
    grid = (m // bm, n // bn)
    return pl.pallas_call(
        _mm_kernel,
        grid=grid,
        in_specs=[pl.BlockSpec((bm, k), lambda i, j: (i, 0)),
                  pl.BlockSpec((k, bn), lambda i, j: (0, j))],
        out_specs=pl.BlockSpec((bm, bn), lambda i, j: (i, j)),
        out_shape=jax.ShapeDtypeStruct((m, n), jnp.float32),
    )(x, w)


def _qp_kernel(q_ref, ket_ref, o_ref):
    o_ref[...] = jnp.dot(q_ref[...], ket_ref[0],
                         preferred_element_type=jnp.float32)[None]


def _qp(q, ket, C, bm=840, bn=1536):
    H = ket.shape[0]
    grid = (H, N // bm, NCP // bn)
    return pl.pallas_call(
        _qp_kernel,
        grid=grid,
        in_specs=[pl.BlockSpec((bm, C), lambda h, i, j: (i, h)),
                  pl.BlockSpec((1, C, bn), lambda h, i, j: (h, 0, j))],
        out_specs=pl.BlockSpec((1, bm, bn), lambda h, i, j: (h, i, j)),
        out_shape=jax.ShapeDtypeStruct((H, N, NCP), jnp.float32),
    )(q, ket)


def _abve_kernel(ab_ref, ve_ref, o_ref):
    @pl.when(pl.program_id(2) == 0)
    def _init():
        o_ref[...] = jnp.zeros_like(o_ref)
    o_ref[...] += jnp.dot(ab_ref[0].astype(jnp.bfloat16), ve_ref[0],
                          preferred_element_type=jnp.float32)[None]


def _abve(ab, ve, C, bm=840, bk=2688):
    H = ab.shape[0]
    grid = (H, N // bm, NCP // bk)
    return pl.pallas_call(
        _abve_kernel,
        grid=grid,
        in_specs=[pl.BlockSpec((1, bm, bk), lambda h, i, k: (h, i, k)),
                  pl.BlockSpec((1, bk, C), lambda h, i, k: (h, k, 0))],
        out_specs=pl.BlockSpec((1, bm, C), lambda h, i, k: (h, i, 0)),
        out_shape=jax.ShapeDtypeStruct((H, N, C), jnp.float32),
    )(ab, ve)


def _layer(x_bf, Wcat_bf, bcat, e_tab, src, dst, eid, idx_k, idx_e, H, C):
    HC = H * C
    qkvs = _mm(x_bf, Wcat_bf, bn=min(2048, 4 * HC)) + bcat
    q = qkvs[:, :HC].astype(jnp.bfloat16)
    k = qkvs[:, HC:2 * HC]
    v = qkvs[:, 2 * HC:3 * HC]
    s = qkvs[:, 3 * HC:]

    k3 = k.reshape(N, H, C).transpose(1, 0, 2)
    v3 = v.reshape(N, H, C).transpose(1, 0, 2)
    e3 = e_tab.reshape(NE, H, C).transpose(1, 0, 2)
    pad = jnp.zeros((H, NCP - NC, C), jnp.float32)
    ke = jnp.concatenate([k3, e3, pad], axis=1).astype(jnp.bfloat16)
    ve = jnp.concatenate([v3, e3, pad], axis=1).astype(jnp.bfloat16)
    ket = ke.transpose(0, 2, 1)

    qp = _qp(q, ket, C)
    qpf = qp.reshape(H, N * NCP)
    scale = np.float32(1.0 / np.sqrt(C))
    alpha = (qpf[:, idx_k] + qpf[:, idx_e]) * scale

    amax = jax.ops.segment_max(alpha.T, dst, num_segments=N)
    amax = jnp.where(jnp.isfinite(amax), amax, 0.0)
    ex = jnp.exp(alpha - amax.T[:, dst])
    den = jax.ops.segment_sum(ex.T, dst, num_segments=N)

    abf = jnp.zeros((H, N * NCP), jnp.float32)
    idx_all = jnp.concatenate([idx_k, idx_e])
    ex2 = jnp.concatenate([ex, ex], axis=1)
    abf = abf.at[:, idx_all].add(ex2)
    ab = abf.reshape(H, N, NCP)

    out = _abve(ab, ve, C)
    out = out / (den.T[:, :, None] + 1e-16)
    out = out.transpose(1, 0, 2).reshape(N, HC)
    return jax.nn.relu(out + s)


def kernel(x, edge_index, edge_features,
           Wq1, bq1, Wk1, bk1, Wv1, bv1, We1, Ws1, bs1,
           Wq2, bq2, Wk2, bk2, Wv2, bv2, We2, Ws2, bs2):
    src = edge_index[0]
    dst = edge_index[1]
    eid = jnp.arange(E, dtype=jnp.int32) % NE
    idx_k = dst * NCP + src
    idx_e = dst * NCP + N + eid

    W1 = jnp.concatenate([Wq1, Wk1, Wv1, Ws1], axis=1).astype(jnp.bfloat16)
    b1 = jnp.concatenate([bq1, bk1, bv1, bs1])
    W2 = jnp.concatenate([Wq2, Wk2, Wv2, Ws2], axis=1).astype(jnp.bfloat16)
    b2 = jnp.concatenate([bq2, bk2, bv2, bs2])
    e_tab1 = edge_features @ We1
    e_tab2 = edge_features @ We2

    h = _layer(x.astype(jnp.bfloat16), W1, b1, e_tab1,
               src, dst, eid, idx_k, idx_e, H=2, C=1024)
    h2 = _layer(h.astype(jnp.bfloat16), W2, b2, e_tab2,
                src, dst, eid, idx_k, idx_e, H=1, C=64)
    return h2.reshape(-1, 420 * 64)
